# baseline (speedup 1.0000x reference)
import functools

import jax
import jax.numpy as jnp
from jax import lax
from jax.experimental import pallas as pl
from jax.experimental.pallas import tpu as pltpu
from jax.experimental.pallas import tpu_sc as plsc

N_NODES = 10000
D = 128
NP = 10240
NC, NS = 2, 16
NTILE = NC * NS
BLK = 128
NBLK = 84
EPT = NBLK * BLK
E_PAD = NTILE * EPT
PAD_SRC = 10016
PAD_DST = 10239
ROWS_PER_TILE = NP // NS
SHIFT = 14
MASK = (1 << SHIFT) - 1

_MESH = plsc.VectorSubcoreMesh(
    core_axis_name="c", subcore_axis_name="s", num_cores=NC, num_subcores=NS)

ROW_BLOCK = 512
_TC_GRID = NP // ROW_BLOCK


def _unpack_src(pk_v, j, out_ref):
    for k in range(8):
        v = pk_v[j, 0, pl.ds(16 * k, 16)]
        out_ref[0, pl.ds(16 * k, 16)] = v & MASK


def _unpack_dst(pk_v, j, out_ref):
    for k in range(8):
        v = pk_v[j, 0, pl.ds(16 * k, 16)]
        out_ref[0, pl.ds(16 * k, 16)] = v >> SHIFT


def _zero_rows(buf, nrows, ncols):
    @pl.loop(0, nrows)
    def _z(i):
        for k in range(ncols // 16):
            buf[i, pl.ds(16 * k, 16)] = jnp.zeros((16,), jnp.float32)


def _zero_acc_slice(buf, acc, s):
    @pl.loop(0, ROWS_PER_TILE // BLK)
    def _zc(t):
        pltpu.sync_copy(buf, acc.at[pl.ds(s * ROWS_PER_TILE + t * BLK, BLK)])


def _deg_body(pk_hbm, deg_hbm, pk_v, ones_v, ds_v, acc, sem):
    c = lax.axis_index("c")
    s = lax.axis_index("s")
    wid = c * NS + s

    _zero_rows(ones_v, BLK, D)
    _zero_acc_slice(ones_v, acc, s)

    @pl.loop(0, BLK)
    def _fill(i):
        for k in range(D // 16):
            ones_v[i, pl.ds(16 * k, 16)] = jnp.ones((16,), jnp.float32)

    pltpu.sync_copy(pk_hbm.at[pl.ds(wid * NBLK, NBLK)], pk_v)
    plsc.subcore_barrier()

    @pl.loop(0, NBLK)
    def _scatter(j):
        _unpack_dst(pk_v, j, ds_v)
        pltpu.sync_copy(ones_v, acc.at[ds_v.at[0]], add=True)

    plsc.subcore_barrier()
    pltpu.sync_copy(acc.at[pl.ds(s * ROWS_PER_TILE, ROWS_PER_TILE)],
                    deg_hbm.at[c].at[pl.ds(s * ROWS_PER_TILE, ROWS_PER_TILE)])


def _sc_degree(pk3):
    kern = pl.kernel(
        _deg_body,
        out_type=jax.ShapeDtypeStruct((NC, NP, D), jnp.float32),
        mesh=_MESH,
        scratch_types=[
            pltpu.VMEM((NBLK, 1, BLK), jnp.int32),
            pltpu.VMEM((BLK, D), jnp.float32),
            pltpu.VMEM((1, BLK), jnp.int32),
            pltpu.VMEM_SHARED((NP, D), jnp.float32),
            pltpu.SemaphoreType.DMA,
        ],
    )
    return kern(pk3)


HBLK = BLK // 2


def _unpack_half_src(pk_v, row, half, out_ref):
    for k in range(HBLK // 16):
        v = pk_v[row, 0, pl.ds(half * HBLK + 16 * k, 16)]
        out_ref[0, pl.ds(16 * k, 16)] = v & MASK


def _unpack_half_dst(pk_v, row, half, out_ref):
    for k in range(HBLK // 16):
        v = pk_v[row, 0, pl.ds(half * HBLK + 16 * k, 16)]
        out_ref[0, pl.ds(16 * k, 16)] = v >> SHIFT


def _segsum_body(y_hbm, pk_hbm, out_hbm,
                 pk_v, ss_v, ds_v, bufs, acc, sems):
    c = lax.axis_index("c")
    s = lax.axis_index("s")
    wid = c * NS + s

    _zero_rows(bufs.at[0], HBLK, D)

    @pl.loop(0, ROWS_PER_TILE // HBLK)
    def _zc(t):
        pltpu.sync_copy(bufs.at[0],
                        acc.at[pl.ds(s * ROWS_PER_TILE + t * HBLK, HBLK)])

    pltpu.sync_copy(pk_hbm.at[pl.ds(wid * NBLK, NBLK)], pk_v)
    plsc.subcore_barrier()

    for t in range(4):
        _unpack_half_src(pk_v, t // 2, t % 2, ss_v.at[t])
        pltpu.async_copy(y_hbm.at[ss_v.at[t, 0]], bufs.at[t], sems.at[t])

    @pl.loop(0, NBLK, step=2)
    def _edge(r):
        for t in range(4):
            row = r + t // 2
            pltpu.make_async_copy(
                y_hbm.at[ss_v.at[t, 0]], bufs.at[t], sems.at[t]).wait()
            _unpack_half_dst(pk_v, row, t % 2, ds_v)
            pltpu.sync_copy(bufs.at[t], acc.at[ds_v.at[0]], add=True)

            @pl.when(row + 2 < NBLK)
            def _pref():
                _unpack_half_src(pk_v, row + 2, t % 2, ss_v.at[t])
                pltpu.async_copy(y_hbm.at[ss_v.at[t, 0]], bufs.at[t],
                                 sems.at[t])

    plsc.subcore_barrier()
    pltpu.sync_copy(acc.at[pl.ds(s * ROWS_PER_TILE, ROWS_PER_TILE)],
                    out_hbm.at[c].at[pl.ds(s * ROWS_PER_TILE, ROWS_PER_TILE)])


def _sc_segsum(y, pk3):
    kern = pl.kernel(
        _segsum_body,
        out_type=jax.ShapeDtypeStruct((NC, NP, D), jnp.float32),
        mesh=_MESH,
        scratch_types=[
            pltpu.VMEM((NBLK, 1, BLK), jnp.int32),
            pltpu.VMEM((4, 1, HBLK), jnp.int32),
            pltpu.VMEM((1, HBLK), jnp.int32),
            pltpu.VMEM((4, HBLK, D), jnp.float32),
            pltpu.VMEM_SHARED((NP, D), jnp.float32),
            pltpu.SemaphoreType.DMA((4,)),
        ],
    )
    return kern(y, pk3)


def _l1_body(deg_ref, x_ref, w_ref, y_ref, dinv_ref):
    deg = deg_ref[...]
    dinv = jnp.where(deg > 0, lax.rsqrt(deg), 0.0)
    dinv_ref[...] = dinv
    y_ref[...] = jnp.dot(x_ref[...], w_ref[...],
                         preferred_element_type=jnp.float32) * dinv


def _tc_layer1(deg_col, x, W):
    return pl.pallas_call(
        _l1_body,
        grid=(_TC_GRID,),
        in_specs=[
            pl.BlockSpec((ROW_BLOCK, 1), lambda i: (i, 0)),
            pl.BlockSpec((ROW_BLOCK, D), lambda i: (i, 0)),
            pl.BlockSpec((D, D), lambda i: (0, 0)),
        ],
        out_specs=[
            pl.BlockSpec((ROW_BLOCK, D), lambda i: (i, 0)),
            pl.BlockSpec((ROW_BLOCK, 1), lambda i: (i, 0)),
        ],
        out_shape=[
            jax.ShapeDtypeStruct((NP, D), jnp.float32),
            jax.ShapeDtypeStruct((NP, 1), jnp.float32),
        ],
    )(deg_col, x, W)


def _fused_body(p_ref, dinv_ref, b_ref, w_ref, y_ref):
    dinv = dinv_ref[...]
    h = jnp.maximum(dinv * (p_ref[0] + p_ref[1]) + b_ref[...], 0.0)
    y_ref[...] = jnp.dot(h, w_ref[...],
                         preferred_element_type=jnp.float32) * dinv


def _tc_fused(parts, dinv_col, b, W):
    return pl.pallas_call(
        _fused_body,
        grid=(_TC_GRID,),
        in_specs=[
            pl.BlockSpec((NC, ROW_BLOCK, D), lambda i: (0, i, 0)),
            pl.BlockSpec((ROW_BLOCK, 1), lambda i: (i, 0)),
            pl.BlockSpec((1, D), lambda i: (0, 0)),
            pl.BlockSpec((D, D), lambda i: (0, 0)),
        ],
        out_specs=pl.BlockSpec((ROW_BLOCK, D), lambda i: (i, 0)),
        out_shape=jax.ShapeDtypeStruct((NP, D), jnp.float32),
    )(parts, dinv_col, b.reshape(1, D), W)


def _epi_body(p_ref, dinv_ref, b_ref, o_ref):
    o_ref[...] = dinv_ref[...] * (p_ref[0] + p_ref[1]) + b_ref[...]


def _tc_epilogue(parts, dinv_col, b):
    return pl.pallas_call(
        _epi_body,
        grid=(_TC_GRID,),
        in_specs=[
            pl.BlockSpec((NC, ROW_BLOCK, D), lambda i: (0, i, 0)),
            pl.BlockSpec((ROW_BLOCK, 1), lambda i: (i, 0)),
            pl.BlockSpec((1, D), lambda i: (0, 0)),
        ],
        out_specs=pl.BlockSpec((ROW_BLOCK, D), lambda i: (i, 0)),
        out_shape=jax.ShapeDtypeStruct((NP, D), jnp.float32),
    )(parts, dinv_col, b.reshape(1, D))


def kernel(x, edge_index, W1, b1, W2, b2, W3, b3):
    n = x.shape[0]
    loops = jnp.arange(n, dtype=edge_index.dtype)
    n_real = edge_index.shape[1] + n
    pad = E_PAD - n_real
    src = jnp.concatenate(
        [edge_index[0], loops, jnp.full((pad,), PAD_SRC, edge_index.dtype)])
    dst = jnp.concatenate(
        [edge_index[1], loops, jnp.full((pad,), PAD_DST, edge_index.dtype)])
    packed = src | (dst << SHIFT)
    pk3 = packed.reshape(E_PAD // BLK, 1, BLK)

    xp = jnp.pad(x, ((0, NP - n), (0, 0)))

    degp = _sc_degree(pk3)
    deg_col = (degp[0] + degp[1])[:, :1]

    y1, dinv_col = _tc_layer1(deg_col, xp, W1)
    s1 = _sc_segsum(y1, pk3)
    y2 = _tc_fused(s1, dinv_col, b1, W2)
    s2 = _sc_segsum(y2, pk3)
    y3 = _tc_fused(s2, dinv_col, b2, W3)
    s3 = _sc_segsum(y3, pk3)
    out = _tc_epilogue(s3, dinv_col, b3)
    return out[:n]


# speedup vs baseline: 3.5723x; 3.5723x over previous
import dataclasses
import functools

import jax
import jax.numpy as jnp
from jax import lax
from jax.experimental import pallas as pl
from jax.experimental.pallas import tpu as pltpu
from jax.experimental.pallas import tpu_sc as plsc

N_NODES = 10000
D = 128
NP = 10240
NC, NS = 2, 16
NTILE = NC * NS
BLK = 128
HBLK = BLK // 2
NBLK = 80
EPT = NBLK * BLK
E_PAD = NTILE * EPT
PAD_SRC = 10016
PAD_DST = 10239
ROWS_PER_TILE = NP // NS
SHIFT = 14
MASK = (1 << SHIFT) - 1

HALF = NP // 2
CAP = 3072
CBLK = CAP // BLK
ACC_ROWS = 6144
ACC_PER_TILE = ACC_ROWS // NS
WB_PER_TILE = HALF // NS
TRASH = HALF

_MESH = plsc.VectorSubcoreMesh(
    core_axis_name="c", subcore_axis_name="s", num_cores=NC, num_subcores=NS)

ROW_BLOCK = 512
_TC_GRID = NP // ROW_BLOCK


def _unpack_dst(pk_v, j, out_ref):
    for k in range(8):
        v = pk_v[j, 0, pl.ds(16 * k, 16)]
        out_ref[0, pl.ds(16 * k, 16)] = v >> SHIFT


def _unpack_half_src(pk_v, row, half, out_ref):
    for k in range(HBLK // 16):
        v = pk_v[pl.ds(row * BLK + half * HBLK + 16 * k, 16)]
        out_ref[0, pl.ds(16 * k, 16)] = v & MASK


def _unpack_half_dst(pk_v, row, half, out_ref):
    for k in range(HBLK // 16):
        v = pk_v[pl.ds(row * BLK + half * HBLK + 16 * k, 16)]
        out_ref[0, pl.ds(16 * k, 16)] = v >> SHIFT


def _zero_rows(buf, nrows, ncols):
    @pl.loop(0, nrows)
    def _z(i):
        for k in range(ncols // 16):
            buf[i, pl.ds(16 * k, 16)] = jnp.zeros((16,), jnp.float32)


def _group_body(pk_hbm, gl_hbm, pk_v, gbuf, sem):
    c = lax.axis_index("c")
    s = lax.axis_index("s")
    wid = c * NS + s

    pltpu.sync_copy(pk_hbm.at[pl.ds(wid * NBLK, NBLK)], pk_v)

    trash_v = jnp.full((16,), TRASH << SHIFT, jnp.int32)

    @pl.loop(0, 4 * CAP // 16)
    def _fill(i):
        gbuf[pl.ds(16 * i, 16)] = trash_v

    @pl.loop(0, NBLK, init_carry=(0, 0, 0, 0))
    def _blk(blk, carry):
        cnts = list(carry)
        for k in range(BLK // 16):
            w = pk_v[blk, 0, pl.ds(16 * k, 16)]
            srcv = w & MASK
            dstv = w >> SHIFT
            keep = srcv != PAD_SRC
            sh = srcv >= HALF
            dh = dstv >= HALF
            gv = jnp.where(sh, 1, 0) + jnp.where(dh, 2, 0)
            lw = ((srcv - jnp.where(sh, HALF, 0))
                  | ((dstv - jnp.where(dh, HALF, 0)) << SHIFT))
            for g in range(4):
                m = keep & (gv == g)
                mi = jnp.where(m, 1, 0)
                r = plsc.cumsum(mi) - mi
                pos = cnts[g] + r
                m = m & (pos < CAP)
                plsc.store_scatter(gbuf, [g * CAP + pos], lw, mask=m)
                cnts[g] = cnts[g] + lax.reduce_sum(mi, axes=(0,))
        return tuple(cnts)

    pltpu.sync_copy(gbuf, gl_hbm.at[wid])


_GROUP_CP = pltpu.CompilerParams()
if "needs_layout_passes" in pltpu.CompilerParams.__dataclass_fields__:
    _GROUP_CP = dataclasses.replace(_GROUP_CP, needs_layout_passes=False)


def _sc_group(pk3):
    kern = pl.kernel(
        _group_body,
        compiler_params=_GROUP_CP,
        out_type=jax.ShapeDtypeStruct((NTILE, 4 * CAP), jnp.int32),
        mesh=_MESH,
        scratch_types=[
            pltpu.VMEM((NBLK, 1, BLK), jnp.int32),
            pltpu.VMEM((4 * CAP,), jnp.int32),
            pltpu.SemaphoreType.DMA,
        ],
    )
    return kern(pk3)


def _deg_body(pk_hbm, deg_hbm, pk_v, ones_v, ds_v, acc, sem):
    c = lax.axis_index("c")
    s = lax.axis_index("s")
    wid = c * NS + s

    _zero_rows(ones_v, BLK, D)

    @pl.loop(0, ROWS_PER_TILE // BLK)
    def _zc(t):
        pltpu.sync_copy(ones_v, acc.at[pl.ds(s * ROWS_PER_TILE + t * BLK, BLK)])

    @pl.loop(0, BLK)
    def _fill(i):
        for k in range(D // 16):
            ones_v[i, pl.ds(16 * k, 16)] = jnp.ones((16,), jnp.float32)

    pltpu.sync_copy(pk_hbm.at[pl.ds(wid * NBLK, NBLK)], pk_v)
    plsc.subcore_barrier()

    @pl.loop(0, NBLK)
    def _scatter(j):
        _unpack_dst(pk_v, j, ds_v)
        pltpu.sync_copy(ones_v, acc.at[ds_v.at[0]], add=True)

    plsc.subcore_barrier()
    pltpu.sync_copy(acc.at[pl.ds(s * ROWS_PER_TILE, ROWS_PER_TILE)],
                    deg_hbm.at[c].at[pl.ds(s * ROWS_PER_TILE, ROWS_PER_TILE)])


def _sc_degree(pk3):
    kern = pl.kernel(
        _deg_body,
        compiler_params=_GROUP_CP,
        out_type=jax.ShapeDtypeStruct((NC, NP, D), jnp.float32),
        mesh=_MESH,
        scratch_types=[
            pltpu.VMEM((NBLK, 1, BLK), jnp.int32),
            pltpu.VMEM((BLK, D), jnp.float32),
            pltpu.VMEM((1, BLK), jnp.int32),
            pltpu.VMEM_SHARED((NP, D), jnp.float32),
            pltpu.SemaphoreType.DMA,
        ],
    )
    return kern(pk3)


def _segsum_body(y_hbm, gl_hbm, out_hbm,
                 pk_v, ss_v, ds_v, bufs, ystage, acc, sems):
    c = lax.axis_index("c")
    s = lax.axis_index("s")

    _zero_rows(bufs.at[0], HBLK, D)

    @pl.loop(0, ACC_PER_TILE // HBLK)
    def _zc(t):
        pltpu.sync_copy(bufs.at[0],
                        acc.at[pl.ds(s * ACC_PER_TILE + t * HBLK, HBLK)])

    for p in range(2):
        pltpu.sync_copy(y_hbm.at[pl.ds(p * HALF + s * WB_PER_TILE,
                                       WB_PER_TILE)],
                        ystage.at[pl.ds(s * WB_PER_TILE, WB_PER_TILE)])
        plsc.subcore_barrier()

        for q in range(2):
            pltpu.sync_copy(gl_hbm.at[2 * s + q].at[pl.ds((p + 2 * c) * CAP, CAP)], pk_v)

            for t in range(4):
                _unpack_half_src(pk_v, t // 2, t % 2, ss_v.at[t])
                pltpu.async_copy(ystage.at[ss_v.at[t, 0]], bufs.at[t],
                                 sems.at[t])

            @pl.loop(0, CBLK, step=2)
            def _edge(r):
                for t in range(4):
                    row = r + t // 2
                    pltpu.make_async_copy(
                        ystage.at[ss_v.at[t, 0]], bufs.at[t],
                        sems.at[t]).wait()
                    _unpack_half_dst(pk_v, row, t % 2, ds_v)
                    pltpu.sync_copy(bufs.at[t], acc.at[ds_v.at[0]], add=True)

                    @pl.when(row + 2 < CBLK)
                    def _pref():
                        _unpack_half_src(pk_v, row + 2, t % 2, ss_v.at[t])
                        pltpu.async_copy(ystage.at[ss_v.at[t, 0]], bufs.at[t],
                                         sems.at[t])

        plsc.subcore_barrier()

    pltpu.sync_copy(acc.at[pl.ds(s * WB_PER_TILE, WB_PER_TILE)],
                    out_hbm.at[pl.ds(c * HALF + s * WB_PER_TILE,
                                     WB_PER_TILE)])


def _sc_segsum(y, gl):
    kern = pl.kernel(
        _segsum_body,
        compiler_params=_GROUP_CP,
        out_type=jax.ShapeDtypeStruct((NP, D), jnp.float32),
        mesh=_MESH,
        scratch_types=[
            pltpu.VMEM((CAP,), jnp.int32),
            pltpu.VMEM((4, 1, HBLK), jnp.int32),
            pltpu.VMEM((1, HBLK), jnp.int32),
            pltpu.VMEM((4, HBLK, D), jnp.float32),
            pltpu.VMEM_SHARED((HALF, D), jnp.float32),
            pltpu.VMEM_SHARED((ACC_ROWS, D), jnp.float32),
            pltpu.SemaphoreType.DMA((4,)),
        ],
    )
    return kern(y, gl)


def _l1_body(deg_ref, x_ref, w_ref, y_ref, dinv_ref):
    dinv = lax.rsqrt(deg_ref[...] + 1.0)
    dinv_ref[...] = dinv
    y_ref[...] = jnp.dot(x_ref[...], w_ref[...],
                         preferred_element_type=jnp.float32) * dinv


def _tc_layer1(deg_col, x, W):
    return pl.pallas_call(
        _l1_body,
        grid=(_TC_GRID,),
        in_specs=[
            pl.BlockSpec((ROW_BLOCK, 1), lambda i: (i, 0)),
            pl.BlockSpec((ROW_BLOCK, D), lambda i: (i, 0)),
            pl.BlockSpec((D, D), lambda i: (0, 0)),
        ],
        out_specs=[
            pl.BlockSpec((ROW_BLOCK, D), lambda i: (i, 0)),
            pl.BlockSpec((ROW_BLOCK, 1), lambda i: (i, 0)),
        ],
        out_shape=[
            jax.ShapeDtypeStruct((NP, D), jnp.float32),
            jax.ShapeDtypeStruct((NP, 1), jnp.float32),
        ],
    )(deg_col, x, W)


def _fused_body(p_ref, yp_ref, dinv_ref, b_ref, w_ref, y_ref):
    dinv = dinv_ref[...]
    h = jnp.maximum(dinv * (p_ref[...] + yp_ref[...]) + b_ref[...], 0.0)
    y_ref[...] = jnp.dot(h, w_ref[...],
                         preferred_element_type=jnp.float32) * dinv


def _tc_fused(part, yprev, dinv_col, b, W):
    return pl.pallas_call(
        _fused_body,
        grid=(_TC_GRID,),
        in_specs=[
            pl.BlockSpec((ROW_BLOCK, D), lambda i: (i, 0)),
            pl.BlockSpec((ROW_BLOCK, D), lambda i: (i, 0)),
            pl.BlockSpec((ROW_BLOCK, 1), lambda i: (i, 0)),
            pl.BlockSpec((1, D), lambda i: (0, 0)),
            pl.BlockSpec((D, D), lambda i: (0, 0)),
        ],
        out_specs=pl.BlockSpec((ROW_BLOCK, D), lambda i: (i, 0)),
        out_shape=jax.ShapeDtypeStruct((NP, D), jnp.float32),
    )(part, yprev, dinv_col, b.reshape(1, D), W)


def _epi_body(p_ref, yp_ref, dinv_ref, b_ref, o_ref):
    o_ref[...] = dinv_ref[...] * (p_ref[...] + yp_ref[...]) + b_ref[...]


def _tc_epilogue(part, yprev, dinv_col, b):
    return pl.pallas_call(
        _epi_body,
        grid=(_TC_GRID,),
        in_specs=[
            pl.BlockSpec((ROW_BLOCK, D), lambda i: (i, 0)),
            pl.BlockSpec((ROW_BLOCK, D), lambda i: (i, 0)),
            pl.BlockSpec((ROW_BLOCK, 1), lambda i: (i, 0)),
            pl.BlockSpec((1, D), lambda i: (0, 0)),
        ],
        out_specs=pl.BlockSpec((ROW_BLOCK, D), lambda i: (i, 0)),
        out_shape=jax.ShapeDtypeStruct((NP, D), jnp.float32),
    )(part, yprev, dinv_col, b.reshape(1, D))


def kernel(x, edge_index, W1, b1, W2, b2, W3, b3):
    n = x.shape[0]
    pad = E_PAD - edge_index.shape[1]
    src = jnp.concatenate(
        [edge_index[0], jnp.full((pad,), PAD_SRC, edge_index.dtype)])
    dst = jnp.concatenate(
        [edge_index[1], jnp.full((pad,), PAD_DST, edge_index.dtype)])
    packed = src | (dst << SHIFT)
    pk3 = packed.reshape(E_PAD // BLK, 1, BLK)

    xp = jnp.pad(x, ((0, NP - n), (0, 0)))

    gl = _sc_group(pk3)
    degp = _sc_degree(pk3)
    deg_col = (degp[0] + degp[1])[:, :1]

    y1, dinv_col = _tc_layer1(deg_col, xp, W1)
    s1 = _sc_segsum(y1, gl)
    y2 = _tc_fused(s1, y1, dinv_col, b1, W2)
    s2 = _sc_segsum(y2, gl)
    y3 = _tc_fused(s2, y2, dinv_col, b2, W3)
    s3 = _sc_segsum(y3, gl)
    out = _tc_epilogue(s3, y3, dinv_col, b3)
    return out[:n]
